# baseline (speedup 1.0000x reference)
import functools

import jax
import jax.numpy as jnp
from jax import lax
from jax.experimental import pallas as pl
from jax.experimental.pallas import tpu as pltpu
from jax.experimental.pallas import tpu_sc as plsc

N = 11776; T = 288; E = 188416; NG = 512
WIN = 32; STR = 16; K = (T - WIN) // STR + 1
H = 4; C = 64; D = H * C; IN = 6; ED = 2; TH = 64; FC = 64; NC = 2
KD = K * D

BN = 4
NBLK = N // BN
NW = 32
BPW = NBLK // NW
CH = 8
WOFF = 128

_R = 512
_RP = 256


def _stats_body(x_ref, o_ref):
    x = x_ref[...]
    cols = []
    for k in range(K):
        w = x[:, k * STR:k * STR + WIN]
        mean = jnp.mean(w, axis=-1)
        mn = jnp.min(w, axis=-1)
        mx = jnp.max(w, axis=-1)
        c = w - mean[:, None]
        c2 = c * c
        var = jnp.mean(c2, axis=-1)
        std = jnp.maximum(jnp.sqrt(var), 1e-8)
        std3 = std * std * std
        skew = jnp.mean(c2 * c, axis=-1) / std3
        kurt = jnp.mean(c2 * c2, axis=-1) / (std3 * std)
        cols.append(jnp.stack([mean, mn, mx, var, skew, kurt], axis=-1))
    o_ref[...] = jnp.concatenate(cols, axis=-1)


def _window_stats(x):
    return pl.pallas_call(
        _stats_body,
        grid=(N // _R,),
        in_specs=[pl.BlockSpec((_R, T), lambda i: (i, 0))],
        out_specs=pl.BlockSpec((_R, K * IN), lambda i: (i, 0)),
        out_shape=jax.ShapeDtypeStruct((N, K * IN), jnp.float32),
    )(x)


def _proj0_body(s_ref, wl_ref, bl_ref, wr_ref, br_ref, xl_ref, xr_ref):
    s = s_ref[...]
    wl = wl_ref[...]; wr = wr_ref[...]
    bl = bl_ref[...]; br = br_ref[...]
    for k in range(K):
        accl = jnp.broadcast_to(bl, (_RP, D))
        accr = jnp.broadcast_to(br, (_RP, D))
        for i in range(IN):
            col = s[:, k * IN + i:k * IN + i + 1]
            accl = accl + col * wl[i:i + 1, :]
            accr = accr + col * wr[i:i + 1, :]
        xl_ref[:, k * D:(k + 1) * D] = accl.astype(jnp.bfloat16)
        xr_ref[:, k * D:(k + 1) * D] = accr.astype(jnp.bfloat16)


def _proj0(stats, W_l, b_l, W_r, b_r):
    return pl.pallas_call(
        _proj0_body,
        grid=(N // _RP,),
        in_specs=[
            pl.BlockSpec((_RP, K * IN), lambda i: (i, 0)),
            pl.BlockSpec((IN, D), lambda i: (0, 0)),
            pl.BlockSpec((1, D), lambda i: (0, 0)),
            pl.BlockSpec((IN, D), lambda i: (0, 0)),
            pl.BlockSpec((1, D), lambda i: (0, 0)),
        ],
        out_specs=[
            pl.BlockSpec((_RP, KD), lambda i: (i, 0)),
            pl.BlockSpec((_RP, KD), lambda i: (i, 0)),
        ],
        out_shape=[
            jax.ShapeDtypeStruct((N, KD), jnp.bfloat16),
            jax.ShapeDtypeStruct((N, KD), jnp.bfloat16),
        ],
    )(stats, W_l, b_l.reshape(1, D), W_r, b_r.reshape(1, D))


_BE = 2048


def _ee_body(ea_ref, w0_ref, w1_ref, e0_ref, e1_ref):
    ea = ea_ref[...]
    w0 = w0_ref[...]; w1 = w1_ref[...]
    a0 = ea[:, 0:1]; a1 = ea[:, 1:2]
    e0_ref[...] = (a0 * w0[0:1, :] + a1 * w0[1:2, :]).astype(jnp.bfloat16)
    e1_ref[...] = (a0 * w1[0:1, :] + a1 * w1[1:2, :]).astype(jnp.bfloat16)


def _edge_feats(ea_s, W_e0, W_e1):
    return pl.pallas_call(
        _ee_body,
        grid=(E // _BE,),
        in_specs=[
            pl.BlockSpec((_BE, ED), lambda i: (i, 0)),
            pl.BlockSpec((ED, D), lambda i: (0, 0)),
            pl.BlockSpec((ED, D), lambda i: (0, 0)),
        ],
        out_specs=[
            pl.BlockSpec((_BE, D), lambda i: (i, 0)),
            pl.BlockSpec((_BE, D), lambda i: (i, 0)),
        ],
        out_shape=[
            jax.ShapeDtypeStruct((E, D), jnp.bfloat16),
            jax.ShapeDtypeStruct((E, D), jnp.bfloat16),
        ],
    )(ea_s, W_e0, W_e1)


def _proj1_body(h_ref, s_ref, p_ref, wl_ref, bl_ref, wr_ref, br_ref,
                x1_ref, xl_ref, xr_ref):
    s = s_ref[...]
    p = p_ref[...]
    wl = wl_ref[...]; wr = wr_ref[...]
    bl = bl_ref[...]; br = br_ref[...]
    for k in range(K):
        x1 = h_ref[:, k * D:(k + 1) * D]
        for i in range(IN):
            x1 = x1 + s[:, k * IN + i:k * IN + i + 1] * p[i:i + 1, :]
        x1_ref[:, k * D:(k + 1) * D] = x1
        xl_ref[:, k * D:(k + 1) * D] = (jnp.dot(x1, wl)
                                        + bl).astype(jnp.bfloat16)
        xr_ref[:, k * D:(k + 1) * D] = (jnp.dot(x1, wr)
                                        + br).astype(jnp.bfloat16)


def _proj1(hgat0, stats, P_res0, W_l1, b_l1, W_r1, b_r1):
    return pl.pallas_call(
        _proj1_body,
        grid=(N // _RP,),
        in_specs=[
            pl.BlockSpec((_RP, KD), lambda i: (i, 0)),
            pl.BlockSpec((_RP, K * IN), lambda i: (i, 0)),
            pl.BlockSpec((IN, D), lambda i: (0, 0)),
            pl.BlockSpec((D, D), lambda i: (0, 0)),
            pl.BlockSpec((1, D), lambda i: (0, 0)),
            pl.BlockSpec((D, D), lambda i: (0, 0)),
            pl.BlockSpec((1, D), lambda i: (0, 0)),
        ],
        out_specs=[
            pl.BlockSpec((_RP, KD), lambda i: (i, 0)),
            pl.BlockSpec((_RP, KD), lambda i: (i, 0)),
            pl.BlockSpec((_RP, KD), lambda i: (i, 0)),
        ],
        out_shape=[
            jax.ShapeDtypeStruct((N, KD), jnp.float32),
            jax.ShapeDtypeStruct((N, KD), jnp.bfloat16),
            jax.ShapeDtypeStruct((N, KD), jnp.bfloat16),
        ],
    )(hgat0, stats, P_res0, W_l1, b_l1.reshape(1, D), W_r1,
      b_r1.reshape(1, D))


def _pool_body(h_ref, x1_ref, b_ref, hg_ref, hg_acc, cnt_acc):
    step = pl.program_id(0)

    @pl.when(step == 0)
    def _init():
        hg_acc[...] = jnp.zeros_like(hg_acc)
        cnt_acc[...] = jnp.zeros_like(cnt_acc)

    x2 = h_ref[...] + x1_ref[...]
    brow = b_ref[0]
    oh = jnp.equal(lax.broadcasted_iota(jnp.int32, (NG, _RP), 0),
                   brow).astype(jnp.float32)
    hg_acc[...] += jnp.dot(oh, x2)
    cnt_acc[...] += jnp.sum(oh, axis=1, keepdims=True)

    @pl.when(step == N // _RP - 1)
    def _fin():
        cnt = jnp.maximum(cnt_acc[...], 1.0)
        hg_ref[...] = hg_acc[...] / cnt


def _pool(hgat1, x1, batch3):
    return pl.pallas_call(
        _pool_body,
        grid=(N // _RP,),
        in_specs=[
            pl.BlockSpec((_RP, KD), lambda i: (i, 0)),
            pl.BlockSpec((_RP, KD), lambda i: (i, 0)),
            pl.BlockSpec((1, 1, _RP), lambda i: (i, 0, 0)),
        ],
        out_specs=pl.BlockSpec((NG, KD), lambda i: (0, 0)),
        out_shape=jax.ShapeDtypeStruct((NG, KD), jnp.float32),
        scratch_shapes=[
            pltpu.VMEM((NG, KD), jnp.float32),
            pltpu.VMEM((NG, 1), jnp.float32),
        ],
    )(hgat1, x1, batch3)


def _elu(x):
    return jnp.where(x > 0, x, jnp.exp(jnp.minimum(x, 0.0)) - 1.0)


def _head_body(hg_ref, wpg_ref, bpg_ref, wih_ref, bih_ref, whh_ref, bhh_ref,
               wv_ref, bv_ref, wu_ref, wpc_ref, bpc_ref, wcls_ref, bcls_ref,
               o_ref):
    wpg = wpg_ref[...]; bpg = bpg_ref[...]
    wih = wih_ref[...]; bih = bih_ref[...]
    whh = whh_ref[...]; bhh = bhh_ref[...]
    embs = [_elu(jnp.dot(hg_ref[:, k * D:(k + 1) * D], wpg) + bpg)
            for k in range(K)]
    h = jnp.zeros((NG, TH), jnp.float32)
    outs = []
    for t in range(K):
        gi = jnp.dot(embs[t], wih) + bih
        gh = jnp.dot(h, whh) + bhh
        r = jax.nn.sigmoid(gi[:, :TH] + gh[:, :TH])
        z = jax.nn.sigmoid(gi[:, TH:2 * TH] + gh[:, TH:2 * TH])
        n = jnp.tanh(gi[:, 2 * TH:] + r * gh[:, 2 * TH:])
        h = (1.0 - z) * n + z * h
        outs.append(h)
    wv = wv_ref[...]; bv = bv_ref[...]; wu = wu_ref[...]
    us = [jnp.dot(jnp.tanh(jnp.dot(outs[t], wv) + bv), wu) for t in range(K)]
    u = jnp.concatenate(us, axis=1)
    umax = jnp.max(u, axis=1, keepdims=True)
    eu = jnp.exp(u - umax)
    al = eu / jnp.sum(eu, axis=1, keepdims=True)
    ctx = jnp.zeros((NG, TH), jnp.float32)
    for t in range(K):
        ctx = ctx + al[:, t:t + 1] * outs[t]
    f = _elu(jnp.dot(ctx, wpc_ref[...]) + bpc_ref[...])
    o_ref[...] = jnp.dot(f, wcls_ref[...]) + bcls_ref[...]


def _head(hg, W_pg, b_pg, W_ih, b_ih, W_hh, b_hh, W_v, b_v, W_u, W_pc, b_pc,
          W_cls, b_cls):
    full = lambda *s: pl.BlockSpec(s, lambda: tuple(0 for _ in s))
    return pl.pallas_call(
        _head_body,
        in_specs=[
            full(NG, KD), full(D, TH), full(1, TH), full(TH, 3 * TH),
            full(1, 3 * TH), full(TH, 3 * TH), full(1, 3 * TH), full(TH, TH),
            full(1, TH), full(TH, 1), full(TH, FC), full(1, FC), full(FC, NC),
            full(1, NC),
        ],
        out_specs=full(NG, NC),
        out_shape=jax.ShapeDtypeStruct((NG, NC), jnp.float32),
    )(hg, W_pg, b_pg.reshape(1, TH), W_ih, b_ih.reshape(1, 3 * TH), W_hh,
      b_hh.reshape(1, 3 * TH), W_v, b_v.reshape(1, TH), W_u, W_pc,
      b_pc.reshape(1, FC), W_cls, b_cls.reshape(1, NC))


def _lane_onehot_f(i):
    return jnp.where(lax.iota(jnp.int32, 16) == i, 1.0, 0.0)


def _unpack2(xi):
    even = plsc.bitcast(lax.shift_left(xi, 16), jnp.float32)
    odd = plsc.bitcast(jnp.bitwise_and(xi, jnp.int32(-65536)), jnp.float32)
    return even, odd


def _pack2(a, b):
    ai = lax.shift_right_logical(plsc.bitcast(a, jnp.int32), 16)
    bi = jnp.bitwise_and(plsc.bitcast(b, jnp.int32), jnp.int32(-65536))
    return jnp.bitwise_or(ai, bi)


def _allsum(v):
    for sh in (8, 4, 2, 1):
        v = v + v[jnp.bitwise_xor(lax.iota(jnp.int32, 16), sh)]
    return v


def _sc_gat_body(xl_hbm, xr_hbm, ee_hbm, src_hbm, dst_hbm, woffs_hbm,
                 att_hbm, bias_hbm, out_hbm,
                 woffs_v, idxA_v, idxB_v, dstA_v, dstB_v, ee_v, xlg_v, xr_v,
                 acc_v, den_v, eesum_v, cnt_v, att_v, bias_v, semA, semB):
    cid = lax.axis_index("c")
    sid = lax.axis_index("s")
    wid = sid * 2 + cid

    pltpu.sync_copy(att_hbm, att_v)
    pltpu.sync_copy(bias_hbm, bias_v)
    pltpu.sync_copy(woffs_hbm.at[wid], woffs_v)

    head_oh = [_lane_onehot_f(hh) for hh in range(H)]
    att_regs = [att_v[pl.ds(16 * j, 16)] for j in range(16)]
    ones16 = jnp.full((16,), 1.0, jnp.float32)

    def _edge_update(row, dl, k, ee_regs):
        base2 = k * (D // 2)
        base = k * D
        xis = [xlg_v[row, pl.ds(base2 + 16 * jj, 16)] for jj in range(8)]
        ris = [xr_v[dl, pl.ds(base2 + 16 * jj, 16)] for jj in range(8)]
        xlf = []
        xrf = []
        for jj in range(8):
            e0, e1 = _unpack2(xis[jj])
            xlf.append(e0)
            xlf.append(e1)
            r0, r1 = _unpack2(ris[jj])
            xrf.append(r0)
            xrf.append(r1)
        ms = []
        for j in range(16):
            m = xlf[j] + xrf[j] + ee_regs[j]
            ms.append(jnp.maximum(m, 0.2 * m) * att_regs[j])
        s_h = [((ms[4 * h] + ms[4 * h + 1]) + (ms[4 * h + 2] + ms[4 * h + 3]))
               for h in range(H)]
        exs = [jnp.exp(_allsum(s_h[hh])) for hh in range(H)]
        dvec = (exs[0] * head_oh[0] + exs[1] * head_oh[1]) \
            + (exs[2] * head_oh[2] + exs[3] * head_oh[3])
        den_v[dl, pl.ds(16 * k, 16)] = den_v[dl, pl.ds(16 * k, 16)] + dvec
        accs = [acc_v[dl, pl.ds(base + 16 * j, 16)] for j in range(16)]
        news = [accs[j] + exs[j // 4] * xlf[j] for j in range(16)]
        for j in range(16):
            acc_v[dl, pl.ds(base + 16 * j, 16)] = news[j]

    def _over_k(fn):
        def _one_k(k, cc):
            fn(k)
            return cc
        lax.fori_loop(0, K, _one_k, 0)

    def _block(b, carry):
        n0 = (wid * BPW + b) * BN
        ovec = woffs_v[pl.ds(b, 16)]
        o0 = ovec[0]
        o1 = ovec[1]

        z16 = jnp.zeros((16,), jnp.float32)

        def _zrow(r, c):
            for j in range(KD // 16):
                acc_v[r, pl.ds(16 * j, 16)] = z16
            for j in range(K):
                den_v[r, pl.ds(16 * j, 16)] = z16
            for j in range(D // 16):
                eesum_v[r, pl.ds(16 * j, 16)] = z16
            cnt_v[r] = z16
            return c
        lax.fori_loop(0, BN, _zrow, 0)

        pltpu.sync_copy(xr_hbm.at[pl.ds(n0, BN)], xr_v)

        c0 = (o0 // CH) * CH
        nch = (o1 - c0 + CH - 1) // CH

        def _stage(cidx, idx_ref, row0, sem, dst_ref):
            ebase = c0 + cidx * CH
            pltpu.sync_copy(src_hbm.at[pl.ds(ebase, CH)], idx_ref)
            pltpu.make_async_copy(xl_hbm.at[idx_ref],
                                  xlg_v.at[pl.ds(row0, CH)], sem).start()
            pltpu.sync_copy(dst_hbm.at[pl.ds(ebase, CH)],
                            dst_ref.at[pl.ds(0, CH)])
            pltpu.sync_copy(ee_hbm.at[pl.ds(ebase, CH)],
                            ee_v.at[pl.ds(row0, CH)])

        def _wait(idx_ref, row0, sem):
            pltpu.make_async_copy(xl_hbm.at[idx_ref],
                                  xlg_v.at[pl.ds(row0, CH)], sem).wait()

        def _compute(cidx, row0, dst_ref):
            ebase = c0 + cidx * CH
            dstv = dst_ref[pl.ds(0, 16)]
            for i in range(CH):
                eidx = ebase + i
                valid = jnp.logical_and(eidx >= o0, eidx < o1)

                @pl.when(valid)
                def _do(i=i):
                    row = row0 + i
                    dl = dstv[i] - n0
                    cnt_v[dl] = cnt_v[dl] + ones16
                    ers = [ee_v[row, pl.ds(16 * jj, 16)]
                           for jj in range(8)]
                    erf = []
                    for jj in range(8):
                        u0, u1 = _unpack2(ers[jj])
                        erf.append(u0)
                        erf.append(u1)
                    olds = [eesum_v[dl, pl.ds(16 * j, 16)]
                            for j in range(D // 16)]
                    for j in range(D // 16):
                        eesum_v[dl, pl.ds(16 * j, 16)] = olds[j] + erf[j]

                    _over_k(lambda k: _edge_update(row, dl, k, erf))

        @pl.when(nch > 0)
        def _pro():
            _stage(0, idxA_v, 0, semA, dstA_v)

        def _pair(cc, c2):
            ca = 2 * cc
            cb = 2 * cc + 1
            cn = 2 * cc + 2

            @pl.when(cb < nch)
            def _pfB():
                _stage(cb, idxB_v, CH, semB, dstB_v)
            _wait(idxA_v, 0, semA)
            _compute(ca, 0, dstA_v)

            @pl.when(cn < nch)
            def _pfA():
                _stage(cn, idxA_v, 0, semA, dstA_v)

            @pl.when(cb < nch)
            def _doB():
                _wait(idxB_v, CH, semB)
                _compute(cb, CH, dstB_v)
            return c2
        lax.fori_loop(0, (nch + 1) // 2, _pair, 0)

        pltpu.sync_copy(xl_hbm.at[pl.ds(n0, BN)], xlg_v.at[pl.ds(0, BN)])

        def _selfi(i, c):
            inv = 1.0 / jnp.maximum(cnt_v[i], 1.0)
            eef = [eesum_v[i, pl.ds(16 * j, 16)] * inv
                   for j in range(D // 16)]

            _over_k(lambda k: _edge_update(i, i, k, eef))
            return c
        lax.fori_loop(0, BN, _selfi, 0)

        def _fin(i, c):
            def _kfin(k, cc):
                dn = den_v[i, pl.ds(16 * k, 16)]
                dh = [_allsum(dn * head_oh[hh]) + 1e-16 for hh in range(H)]
                avs = [acc_v[i, pl.ds(k * D + 16 * j, 16)] for j in range(16)]
                bvs = [bias_v[pl.ds(16 * j, 16)] for j in range(16)]
                vs = [avs[j] / dh[j // 4] + bvs[j] for j in range(16)]
                outs = [jnp.where(v > 0, v,
                                  jnp.exp(jnp.minimum(v, 0.0)) - 1.0)
                        for v in vs]
                for j in range(16):
                    acc_v[i, pl.ds(k * D + 16 * j, 16)] = outs[j]
                return cc
            lax.fori_loop(0, K, _kfin, 0)
            return c
        lax.fori_loop(0, BN, _fin, 0)
        pltpu.sync_copy(acc_v, out_hbm.at[pl.ds(n0, BN)])
        return carry

    lax.fori_loop(0, BPW, _block, 0)


def _sc_gat(xl, xr, ee, src_s, dst_s, woffs, att, bias):
    mesh = plsc.VectorSubcoreMesh(core_axis_name="c", subcore_axis_name="s")
    f = functools.partial(
        pl.kernel,
        out_type=jax.ShapeDtypeStruct((N, KD), jnp.float32),
        mesh=mesh,
        compiler_params=pltpu.CompilerParams(needs_layout_passes=False),
        scratch_types=[
            pltpu.VMEM((WOFF,), jnp.int32),
            pltpu.VMEM((CH,), jnp.int32),
            pltpu.VMEM((CH,), jnp.int32),
            pltpu.VMEM((16,), jnp.int32),
            pltpu.VMEM((16,), jnp.int32),
            pltpu.VMEM((2 * CH, D // 2), jnp.int32),
            pltpu.VMEM((2 * CH, KD // 2), jnp.int32),
            pltpu.VMEM((BN, KD // 2), jnp.int32),
            pltpu.VMEM((BN, KD), jnp.float32),
            pltpu.VMEM((BN, 16 * K), jnp.float32),
            pltpu.VMEM((BN, D), jnp.float32),
            pltpu.VMEM((BN, 16), jnp.float32),
            pltpu.VMEM((D,), jnp.float32),
            pltpu.VMEM((D,), jnp.float32),
            pltpu.SemaphoreType.DMA,
            pltpu.SemaphoreType.DMA,
        ],
    )(_sc_gat_wrapped)
    return f(xl, xr, ee, src_s, dst_s, woffs, att, bias)


def _sc_gat_wrapped(xl_hbm, xr_hbm, ee_hbm, src_hbm, dst_hbm, woffs_hbm,
                    att_hbm, bias_hbm, out_hbm, *scratch):
    _sc_gat_body(xl_hbm, xr_hbm, ee_hbm, src_hbm, dst_hbm, woffs_hbm,
                 att_hbm, bias_hbm, out_hbm, *scratch)


def kernel(x, edge_index, edge_attr, batch, W_l0, b_l0, W_r0, b_r0, W_e0,
           att0, bias0, W_l1, b_l1, W_r1, b_r1, W_e1, att1, bias1, P_res0,
           W_pg, b_pg, W_ih, b_ih, W_hh, b_hh, W_v, b_v, W_u, W_pc, b_pc,
           W_cls, b_cls):
    src, dst = edge_index[0], edge_index[1]
    order = jnp.argsort(dst)
    src_s = src[order]
    dst_s = dst[order]
    ea_s = edge_attr[order]
    offs = jnp.searchsorted(
        dst_s, jnp.arange(0, N + 1, BN, dtype=jnp.int32)).astype(jnp.int32)
    widx = jnp.minimum(
        jnp.arange(NW)[:, None] * BPW + jnp.arange(WOFF)[None, :], NBLK)
    woffs = offs[widx]

    def _ilv(w):
        sh = w.shape
        return (w.reshape(sh[:-1] + (D // 32, 2, 16))
                .swapaxes(-2, -1).reshape(sh))

    def _as32(a):
        return lax.bitcast_convert_type(
            a.reshape(a.shape[0], a.shape[1] // 2, 2), jnp.int32)

    stats = _window_stats(x)
    xl0, xr0 = _proj0(stats, _ilv(W_l0), _ilv(b_l0), _ilv(W_r0), _ilv(b_r0))
    ee0, ee1 = _edge_feats(ea_s, _ilv(W_e0), _ilv(W_e1))
    hgat0 = _sc_gat(_as32(xl0), _as32(xr0), _as32(ee0), src_s, dst_s, woffs,
                    att0.reshape(D), bias0)
    x1, xl1, xr1 = _proj1(hgat0, stats, P_res0, _ilv(W_l1), _ilv(b_l1),
                          _ilv(W_r1), _ilv(b_r1))
    hgat1 = _sc_gat(_as32(xl1), _as32(xr1), _as32(ee1), src_s, dst_s, woffs,
                    att1.reshape(D), bias1)
    batch3 = batch.reshape(N // _RP, 1, _RP)
    hg = _pool(hgat1, x1, batch3)
    return _head(hg, W_pg, b_pg, W_ih, b_ih, W_hh, b_hh, W_v, b_v, W_u,
                 W_pc, b_pc, W_cls, b_cls)


# speedup vs baseline: 1.2562x; 1.2562x over previous
import functools

import jax
import jax.numpy as jnp
from jax import lax
from jax.experimental import pallas as pl
from jax.experimental.pallas import tpu as pltpu
from jax.experimental.pallas import tpu_sc as plsc

N = 11776; T = 288; E = 188416; NG = 512
WIN = 32; STR = 16; K = (T - WIN) // STR + 1
H = 4; C = 64; D = H * C; IN = 6; ED = 2; TH = 64; FC = 64; NC = 2
KD = K * D

BN = 4
NBLK = N // BN
NW = 32
BPW = NBLK // NW
CH = 8
WOFF = 128

_R = 512
_RP = 256


def _stats_body(x_ref, o_ref):
    x = x_ref[...]
    cols = []
    for k in range(K):
        w = x[:, k * STR:k * STR + WIN]
        mean = jnp.mean(w, axis=-1)
        mn = jnp.min(w, axis=-1)
        mx = jnp.max(w, axis=-1)
        c = w - mean[:, None]
        c2 = c * c
        var = jnp.mean(c2, axis=-1)
        std = jnp.maximum(jnp.sqrt(var), 1e-8)
        std3 = std * std * std
        skew = jnp.mean(c2 * c, axis=-1) / std3
        kurt = jnp.mean(c2 * c2, axis=-1) / (std3 * std)
        cols.append(jnp.stack([mean, mn, mx, var, skew, kurt], axis=-1))
    o_ref[...] = jnp.concatenate(cols, axis=-1)


def _window_stats(x):
    return pl.pallas_call(
        _stats_body,
        grid=(N // _R,),
        in_specs=[pl.BlockSpec((_R, T), lambda i: (i, 0))],
        out_specs=pl.BlockSpec((_R, K * IN), lambda i: (i, 0)),
        out_shape=jax.ShapeDtypeStruct((N, K * IN), jnp.float32),
    )(x)


def _proj0_body(s_ref, wl_ref, bl_ref, wr_ref, br_ref, xl_ref, xr_ref):
    s = s_ref[...]
    wl = wl_ref[...]; wr = wr_ref[...]
    bl = bl_ref[...]; br = br_ref[...]
    for k in range(K):
        accl = jnp.broadcast_to(bl, (_RP, D))
        accr = jnp.broadcast_to(br, (_RP, D))
        for i in range(IN):
            col = s[:, k * IN + i:k * IN + i + 1]
            accl = accl + col * wl[i:i + 1, :]
            accr = accr + col * wr[i:i + 1, :]
        xl_ref[:, k * D:(k + 1) * D] = accl
        xr_ref[:, k * D:(k + 1) * D] = accr


def _proj0(stats, W_l, b_l, W_r, b_r):
    return pl.pallas_call(
        _proj0_body,
        grid=(N // _RP,),
        in_specs=[
            pl.BlockSpec((_RP, K * IN), lambda i: (i, 0)),
            pl.BlockSpec((IN, D), lambda i: (0, 0)),
            pl.BlockSpec((1, D), lambda i: (0, 0)),
            pl.BlockSpec((IN, D), lambda i: (0, 0)),
            pl.BlockSpec((1, D), lambda i: (0, 0)),
        ],
        out_specs=[
            pl.BlockSpec((_RP, KD), lambda i: (i, 0)),
            pl.BlockSpec((_RP, KD), lambda i: (i, 0)),
        ],
        out_shape=[
            jax.ShapeDtypeStruct((N, KD), jnp.float32),
            jax.ShapeDtypeStruct((N, KD), jnp.float32),
        ],
    )(stats, W_l, b_l.reshape(1, D), W_r, b_r.reshape(1, D))


_BE = 2048


def _ee_body(ea_ref, w0_ref, w1_ref, e0_ref, e1_ref):
    ea = ea_ref[...]
    w0 = w0_ref[...]; w1 = w1_ref[...]
    a0 = ea[:, 0:1]; a1 = ea[:, 1:2]
    e0_ref[...] = a0 * w0[0:1, :] + a1 * w0[1:2, :]
    e1_ref[...] = a0 * w1[0:1, :] + a1 * w1[1:2, :]


def _edge_feats(ea_s, W_e0, W_e1):
    return pl.pallas_call(
        _ee_body,
        grid=(E // _BE,),
        in_specs=[
            pl.BlockSpec((_BE, ED), lambda i: (i, 0)),
            pl.BlockSpec((ED, D), lambda i: (0, 0)),
            pl.BlockSpec((ED, D), lambda i: (0, 0)),
        ],
        out_specs=[
            pl.BlockSpec((_BE, D), lambda i: (i, 0)),
            pl.BlockSpec((_BE, D), lambda i: (i, 0)),
        ],
        out_shape=[
            jax.ShapeDtypeStruct((E, D), jnp.float32),
            jax.ShapeDtypeStruct((E, D), jnp.float32),
        ],
    )(ea_s, W_e0, W_e1)


def _proj1_body(h_ref, s_ref, p_ref, wl_ref, bl_ref, wr_ref, br_ref,
                x1_ref, xl_ref, xr_ref):
    s = s_ref[...]
    p = p_ref[...]
    wl = wl_ref[...]; wr = wr_ref[...]
    bl = bl_ref[...]; br = br_ref[...]
    for k in range(K):
        x1 = h_ref[:, k * D:(k + 1) * D]
        for i in range(IN):
            x1 = x1 + s[:, k * IN + i:k * IN + i + 1] * p[i:i + 1, :]
        x1_ref[:, k * D:(k + 1) * D] = x1
        xl_ref[:, k * D:(k + 1) * D] = jnp.dot(x1, wl) + bl
        xr_ref[:, k * D:(k + 1) * D] = jnp.dot(x1, wr) + br


def _proj1(hgat0, stats, P_res0, W_l1, b_l1, W_r1, b_r1):
    return pl.pallas_call(
        _proj1_body,
        grid=(N // _RP,),
        in_specs=[
            pl.BlockSpec((_RP, KD), lambda i: (i, 0)),
            pl.BlockSpec((_RP, K * IN), lambda i: (i, 0)),
            pl.BlockSpec((IN, D), lambda i: (0, 0)),
            pl.BlockSpec((D, D), lambda i: (0, 0)),
            pl.BlockSpec((1, D), lambda i: (0, 0)),
            pl.BlockSpec((D, D), lambda i: (0, 0)),
            pl.BlockSpec((1, D), lambda i: (0, 0)),
        ],
        out_specs=[
            pl.BlockSpec((_RP, KD), lambda i: (i, 0)),
            pl.BlockSpec((_RP, KD), lambda i: (i, 0)),
            pl.BlockSpec((_RP, KD), lambda i: (i, 0)),
        ],
        out_shape=[
            jax.ShapeDtypeStruct((N, KD), jnp.float32),
            jax.ShapeDtypeStruct((N, KD), jnp.float32),
            jax.ShapeDtypeStruct((N, KD), jnp.float32),
        ],
    )(hgat0, stats, P_res0, W_l1, b_l1.reshape(1, D), W_r1,
      b_r1.reshape(1, D))


def _pool_body(h_ref, x1_ref, b_ref, hg_ref, hg_acc, cnt_acc):
    step = pl.program_id(0)

    @pl.when(step == 0)
    def _init():
        hg_acc[...] = jnp.zeros_like(hg_acc)
        cnt_acc[...] = jnp.zeros_like(cnt_acc)

    x2 = h_ref[...] + x1_ref[...]
    brow = b_ref[0]
    oh = jnp.equal(lax.broadcasted_iota(jnp.int32, (NG, _RP), 0),
                   brow).astype(jnp.float32)
    hg_acc[...] += jnp.dot(oh, x2)
    cnt_acc[...] += jnp.sum(oh, axis=1, keepdims=True)

    @pl.when(step == N // _RP - 1)
    def _fin():
        cnt = jnp.maximum(cnt_acc[...], 1.0)
        hg_ref[...] = hg_acc[...] / cnt


def _pool(hgat1, x1, batch3):
    return pl.pallas_call(
        _pool_body,
        grid=(N // _RP,),
        in_specs=[
            pl.BlockSpec((_RP, KD), lambda i: (i, 0)),
            pl.BlockSpec((_RP, KD), lambda i: (i, 0)),
            pl.BlockSpec((1, 1, _RP), lambda i: (i, 0, 0)),
        ],
        out_specs=pl.BlockSpec((NG, KD), lambda i: (0, 0)),
        out_shape=jax.ShapeDtypeStruct((NG, KD), jnp.float32),
        scratch_shapes=[
            pltpu.VMEM((NG, KD), jnp.float32),
            pltpu.VMEM((NG, 1), jnp.float32),
        ],
    )(hgat1, x1, batch3)


def _elu(x):
    return jnp.where(x > 0, x, jnp.exp(jnp.minimum(x, 0.0)) - 1.0)


def _head_body(hg_ref, wpg_ref, bpg_ref, wih_ref, bih_ref, whh_ref, bhh_ref,
               wv_ref, bv_ref, wu_ref, wpc_ref, bpc_ref, wcls_ref, bcls_ref,
               o_ref):
    wpg = wpg_ref[...]; bpg = bpg_ref[...]
    wih = wih_ref[...]; bih = bih_ref[...]
    whh = whh_ref[...]; bhh = bhh_ref[...]
    embs = [_elu(jnp.dot(hg_ref[:, k * D:(k + 1) * D], wpg) + bpg)
            for k in range(K)]
    h = jnp.zeros((NG, TH), jnp.float32)
    outs = []
    for t in range(K):
        gi = jnp.dot(embs[t], wih) + bih
        gh = jnp.dot(h, whh) + bhh
        r = jax.nn.sigmoid(gi[:, :TH] + gh[:, :TH])
        z = jax.nn.sigmoid(gi[:, TH:2 * TH] + gh[:, TH:2 * TH])
        n = jnp.tanh(gi[:, 2 * TH:] + r * gh[:, 2 * TH:])
        h = (1.0 - z) * n + z * h
        outs.append(h)
    wv = wv_ref[...]; bv = bv_ref[...]; wu = wu_ref[...]
    us = [jnp.dot(jnp.tanh(jnp.dot(outs[t], wv) + bv), wu) for t in range(K)]
    u = jnp.concatenate(us, axis=1)
    umax = jnp.max(u, axis=1, keepdims=True)
    eu = jnp.exp(u - umax)
    al = eu / jnp.sum(eu, axis=1, keepdims=True)
    ctx = jnp.zeros((NG, TH), jnp.float32)
    for t in range(K):
        ctx = ctx + al[:, t:t + 1] * outs[t]
    f = _elu(jnp.dot(ctx, wpc_ref[...]) + bpc_ref[...])
    o_ref[...] = jnp.dot(f, wcls_ref[...]) + bcls_ref[...]


def _head(hg, W_pg, b_pg, W_ih, b_ih, W_hh, b_hh, W_v, b_v, W_u, W_pc, b_pc,
          W_cls, b_cls):
    full = lambda *s: pl.BlockSpec(s, lambda: tuple(0 for _ in s))
    return pl.pallas_call(
        _head_body,
        in_specs=[
            full(NG, KD), full(D, TH), full(1, TH), full(TH, 3 * TH),
            full(1, 3 * TH), full(TH, 3 * TH), full(1, 3 * TH), full(TH, TH),
            full(1, TH), full(TH, 1), full(TH, FC), full(1, FC), full(FC, NC),
            full(1, NC),
        ],
        out_specs=full(NG, NC),
        out_shape=jax.ShapeDtypeStruct((NG, NC), jnp.float32),
    )(hg, W_pg, b_pg.reshape(1, TH), W_ih, b_ih.reshape(1, 3 * TH), W_hh,
      b_hh.reshape(1, 3 * TH), W_v, b_v.reshape(1, TH), W_u, W_pc,
      b_pc.reshape(1, FC), W_cls, b_cls.reshape(1, NC))


def _lane_onehot_f(i):
    return jnp.where(lax.iota(jnp.int32, 16) == i, 1.0, 0.0)


def _allsum(v):
    for sh in (8, 4, 2, 1):
        v = v + v[jnp.bitwise_xor(lax.iota(jnp.int32, 16), sh)]
    return v


def _sc_gat_body(xl_hbm, xr_hbm, ee_hbm, src_hbm, dst_hbm, woffs_hbm,
                 att_hbm, bias_hbm, out_hbm,
                 woffs_v, idxA_v, idxB_v, dstA_v, dstB_v, ee_v, xlg_v, xr_v,
                 acc_v, den_v, eesum_v, cnt_v, att_v, bias_v, semA, semB):
    cid = lax.axis_index("c")
    sid = lax.axis_index("s")
    wid = sid * 2 + cid

    pltpu.sync_copy(att_hbm, att_v)
    pltpu.sync_copy(bias_hbm, bias_v)
    pltpu.sync_copy(woffs_hbm.at[wid], woffs_v)

    head_oh = [_lane_onehot_f(hh) for hh in range(H)]
    att_regs = [att_v[pl.ds(16 * j, 16)] for j in range(16)]
    ones16 = jnp.full((16,), 1.0, jnp.float32)

    def _edge_update(row, dl, k, ee_regs):
        base = k * D
        xls = [xlg_v[row, pl.ds(base + 16 * j, 16)] for j in range(16)]
        xrs = [xr_v[dl, pl.ds(base + 16 * j, 16)] for j in range(16)]
        ms = []
        for j in range(16):
            m = xls[j] + xrs[j] + ee_regs[j]
            ms.append(jnp.maximum(m, 0.2 * m) * att_regs[j])
        s_h = [((ms[4 * h] + ms[4 * h + 1]) + (ms[4 * h + 2] + ms[4 * h + 3]))
               for h in range(H)]
        exs = [jnp.exp(_allsum(s_h[hh])) for hh in range(H)]
        dvec = (exs[0] * head_oh[0] + exs[1] * head_oh[1]) \
            + (exs[2] * head_oh[2] + exs[3] * head_oh[3])
        den_v[dl, pl.ds(16 * k, 16)] = den_v[dl, pl.ds(16 * k, 16)] + dvec
        accs = [acc_v[dl, pl.ds(base + 16 * j, 16)] for j in range(16)]
        news = [accs[j] + exs[j // 4] * xls[j] for j in range(16)]
        for j in range(16):
            acc_v[dl, pl.ds(base + 16 * j, 16)] = news[j]

    def _over_k(fn):
        def _one_k(k, cc):
            fn(k)
            return cc
        lax.fori_loop(0, K, _one_k, 0)

    def _block(b, carry):
        n0 = (wid * BPW + b) * BN
        ovec = woffs_v[pl.ds(b, 16)]
        o0 = ovec[0]
        o1 = ovec[1]

        z16 = jnp.zeros((16,), jnp.float32)

        def _zrow(r, c):
            for j in range(KD // 16):
                acc_v[r, pl.ds(16 * j, 16)] = z16
            for j in range(K):
                den_v[r, pl.ds(16 * j, 16)] = z16
            for j in range(D // 16):
                eesum_v[r, pl.ds(16 * j, 16)] = z16
            cnt_v[r] = z16
            return c
        lax.fori_loop(0, BN, _zrow, 0)

        pltpu.sync_copy(xr_hbm.at[pl.ds(n0, BN)], xr_v)

        c0 = (o0 // CH) * CH
        nch = (o1 - c0 + CH - 1) // CH

        def _stage(cidx, idx_ref, row0, sem, dst_ref):
            ebase = c0 + cidx * CH
            pltpu.sync_copy(src_hbm.at[pl.ds(ebase, CH)], idx_ref)
            pltpu.make_async_copy(xl_hbm.at[idx_ref],
                                  xlg_v.at[pl.ds(row0, CH)], sem).start()
            pltpu.sync_copy(dst_hbm.at[pl.ds(ebase, CH)],
                            dst_ref.at[pl.ds(0, CH)])
            pltpu.sync_copy(ee_hbm.at[pl.ds(ebase, CH)],
                            ee_v.at[pl.ds(row0, CH)])

        def _wait(idx_ref, row0, sem):
            pltpu.make_async_copy(xl_hbm.at[idx_ref],
                                  xlg_v.at[pl.ds(row0, CH)], sem).wait()

        def _compute(cidx, row0, dst_ref):
            ebase = c0 + cidx * CH
            dstv = dst_ref[pl.ds(0, 16)]
            for i in range(CH):
                eidx = ebase + i
                valid = jnp.logical_and(eidx >= o0, eidx < o1)

                @pl.when(valid)
                def _do(i=i):
                    row = row0 + i
                    dl = dstv[i] - n0
                    cnt_v[dl] = cnt_v[dl] + ones16
                    ers = [ee_v[row, pl.ds(16 * j, 16)]
                           for j in range(D // 16)]
                    olds = [eesum_v[dl, pl.ds(16 * j, 16)]
                            for j in range(D // 16)]
                    for j in range(D // 16):
                        eesum_v[dl, pl.ds(16 * j, 16)] = olds[j] + ers[j]

                    _over_k(lambda k: _edge_update(row, dl, k, ers))

        @pl.when(nch > 0)
        def _pro():
            _stage(0, idxA_v, 0, semA, dstA_v)

        def _pair(cc, c2):
            ca = 2 * cc
            cb = 2 * cc + 1
            cn = 2 * cc + 2

            @pl.when(cb < nch)
            def _pfB():
                _stage(cb, idxB_v, CH, semB, dstB_v)
            _wait(idxA_v, 0, semA)
            _compute(ca, 0, dstA_v)

            @pl.when(cn < nch)
            def _pfA():
                _stage(cn, idxA_v, 0, semA, dstA_v)

            @pl.when(cb < nch)
            def _doB():
                _wait(idxB_v, CH, semB)
                _compute(cb, CH, dstB_v)
            return c2
        lax.fori_loop(0, (nch + 1) // 2, _pair, 0)

        pltpu.sync_copy(xl_hbm.at[pl.ds(n0, BN)], xlg_v.at[pl.ds(0, BN)])

        def _selfi(i, c):
            inv = 1.0 / jnp.maximum(cnt_v[i], 1.0)
            ee_regs = [eesum_v[i, pl.ds(16 * j, 16)] * inv
                       for j in range(D // 16)]

            _over_k(lambda k: _edge_update(i, i, k, ee_regs))
            return c
        lax.fori_loop(0, BN, _selfi, 0)

        def _fin(i, c):
            def _kfin(k, cc):
                dn = den_v[i, pl.ds(16 * k, 16)]
                dh = [_allsum(dn * head_oh[hh]) + 1e-16 for hh in range(H)]
                avs = [acc_v[i, pl.ds(k * D + 16 * j, 16)] for j in range(16)]
                bvs = [bias_v[pl.ds(16 * j, 16)] for j in range(16)]
                vs = [avs[j] / dh[j // 4] + bvs[j] for j in range(16)]
                outs = [jnp.where(v > 0, v,
                                  jnp.exp(jnp.minimum(v, 0.0)) - 1.0)
                        for v in vs]
                for j in range(16):
                    acc_v[i, pl.ds(k * D + 16 * j, 16)] = outs[j]
                return cc
            lax.fori_loop(0, K, _kfin, 0)
            return c
        lax.fori_loop(0, BN, _fin, 0)
        pltpu.sync_copy(acc_v, out_hbm.at[pl.ds(n0, BN)])
        return carry

    lax.fori_loop(0, BPW, _block, 0)


def _sc_gat(xl, xr, ee, src_s, dst_s, woffs, att, bias):
    mesh = plsc.VectorSubcoreMesh(core_axis_name="c", subcore_axis_name="s")
    f = functools.partial(
        pl.kernel,
        out_type=jax.ShapeDtypeStruct((N, KD), jnp.float32),
        mesh=mesh,
        scratch_types=[
            pltpu.VMEM((WOFF,), jnp.int32),
            pltpu.VMEM((CH,), jnp.int32),
            pltpu.VMEM((CH,), jnp.int32),
            pltpu.VMEM((16,), jnp.int32),
            pltpu.VMEM((16,), jnp.int32),
            pltpu.VMEM((2 * CH, D), jnp.float32),
            pltpu.VMEM((2 * CH, KD), jnp.float32),
            pltpu.VMEM((BN, KD), jnp.float32),
            pltpu.VMEM((BN, KD), jnp.float32),
            pltpu.VMEM((BN, 16 * K), jnp.float32),
            pltpu.VMEM((BN, D), jnp.float32),
            pltpu.VMEM((BN, 16), jnp.float32),
            pltpu.VMEM((D,), jnp.float32),
            pltpu.VMEM((D,), jnp.float32),
            pltpu.SemaphoreType.DMA,
            pltpu.SemaphoreType.DMA,
        ],
    )(_sc_gat_wrapped)
    return f(xl, xr, ee, src_s, dst_s, woffs, att, bias)


def _sc_gat_wrapped(xl_hbm, xr_hbm, ee_hbm, src_hbm, dst_hbm, woffs_hbm,
                    att_hbm, bias_hbm, out_hbm, *scratch):
    _sc_gat_body(xl_hbm, xr_hbm, ee_hbm, src_hbm, dst_hbm, woffs_hbm,
                 att_hbm, bias_hbm, out_hbm, *scratch)


def kernel(x, edge_index, edge_attr, batch, W_l0, b_l0, W_r0, b_r0, W_e0,
           att0, bias0, W_l1, b_l1, W_r1, b_r1, W_e1, att1, bias1, P_res0,
           W_pg, b_pg, W_ih, b_ih, W_hh, b_hh, W_v, b_v, W_u, W_pc, b_pc,
           W_cls, b_cls):
    src, dst = edge_index[0], edge_index[1]
    order = jnp.argsort(dst)
    src_s = src[order]
    dst_s = dst[order]
    ea_s = edge_attr[order]
    offs = jnp.searchsorted(
        dst_s, jnp.arange(0, N + 1, BN, dtype=jnp.int32)).astype(jnp.int32)
    widx = jnp.minimum(
        jnp.arange(NW)[:, None] * BPW + jnp.arange(WOFF)[None, :], NBLK)
    woffs = offs[widx]

    stats = _window_stats(x)
    xl0, xr0 = _proj0(stats, W_l0, b_l0, W_r0, b_r0)
    ee0, ee1 = _edge_feats(ea_s, W_e0, W_e1)
    hgat0 = _sc_gat(xl0, xr0, ee0, src_s, dst_s, woffs,
                    att0.reshape(D), bias0)
    x1, xl1, xr1 = _proj1(hgat0, stats, P_res0, W_l1, b_l1, W_r1, b_r1)
    hgat1 = _sc_gat(xl1, xr1, ee1, src_s, dst_s, woffs,
                    att1.reshape(D), bias1)
    batch3 = batch.reshape(N // _RP, 1, _RP)
    hg = _pool(hgat1, x1, batch3)
    return _head(hg, W_pg, b_pg, W_ih, b_ih, W_hh, b_hh, W_v, b_v, W_u,
                 W_pc, b_pc, W_cls, b_cls)
